# baseline (speedup 1.0000x reference)
import jax
import jax.numpy as jnp
import numpy as np
from jax.experimental import pallas as pl

N = 10000
E = 320000
N_ELEM = 10
HID = 128
N_BESSEL = 8
N_SH = 9
R_MAX = 5.0
AVG_NEIGH = 32.0


def _geom_kernel(vec_ref, ea_ref, ef_ref):
    v = vec_ref[...]
    x, y, z = v[0:1, :], v[1:2, :], v[2:3, :]
    l2 = x * x + y * y + z * z
    lengths = jnp.sqrt(l2)
    inv = 1.0 / (lengths + 1e-9)
    ux, uy, uz = x * inv, y * inv, z * inv
    ea = jnp.concatenate([
        jnp.full_like(ux, 0.28209479177387814),
        0.4886025119029199 * uy,
        0.4886025119029199 * uz,
        0.4886025119029199 * ux,
        1.0925484305920792 * ux * uy,
        1.0925484305920792 * uy * uz,
        0.31539156525252005 * (3.0 * uz * uz - 1.0),
        1.0925484305920792 * ux * uz,
        0.5462742152960396 * (ux * ux - uy * uy),
    ], axis=0)
    ea_ref[...] = ea
    pref = np.sqrt(2.0 / R_MAX)
    bess = jnp.concatenate(
        [pref * jnp.sin(float(k) * np.pi / R_MAX * lengths) * inv
         for k in range(1, N_BESSEL + 1)], axis=0)
    u_c = jnp.clip(lengths / R_MAX, 0.0, 1.0)
    p = 6.0
    env = (1.0 - ((p + 1.0) * (p + 2.0) / 2.0) * u_c ** 6
           + p * (p + 2.0) * u_c ** 7 - (p * (p + 1.0) / 2.0) * u_c ** 8)
    env = env * (u_c < 1.0).astype(jnp.float32)
    ef_ref[...] = bess * env


def kernel(atom_pos, node_attrs, edge_index, shifts, W_embed, W_radial_0, W_radial_1, W_msg_0, W_msg_1, W_sc_0, W_sc_1, W_prod_0, W_prod_1):
    src = edge_index[0]
    dst = edge_index[1]
    vectors = (atom_pos[dst] - atom_pos[src] + shifts).T
    BE = 16000
    edge_attrs_t, edge_feats_t = pl.pallas_call(
        _geom_kernel,
        grid=(E // BE,),
        in_specs=[pl.BlockSpec((3, BE), lambda i: (0, i))],
        out_specs=[pl.BlockSpec((N_SH, BE), lambda i: (0, i)),
                   pl.BlockSpec((N_BESSEL, BE), lambda i: (0, i))],
        out_shape=[jax.ShapeDtypeStruct((N_SH, E), jnp.float32),
                   jax.ShapeDtypeStruct((N_BESSEL, E), jnp.float32)],
    )(vectors)
    edge_attrs = edge_attrs_t.T
    edge_feats = edge_feats_t.T
    node_feats = node_attrs @ W_embed
    node_feats_list = []
    for (W_radial, W_msg, W_sc, W_prod) in ((W_radial_0, W_msg_0, W_sc_0, W_prod_0), (W_radial_1, W_msg_1, W_sc_1, W_prod_1)):
        radial_w = jnp.tanh(edge_feats @ W_radial)
        msg = node_feats[src] * radial_w
        comps = [jax.ops.segment_sum(msg * edge_attrs[:, c:c + 1], dst, num_segments=N) for c in range(N_SH)]
        node_eq = jnp.concatenate(comps, axis=-1) / AVG_NEIGH
        sc = jnp.einsum('ni,iko,nk->no', node_attrs, W_sc, node_feats)
        node_feats = (node_eq @ W_msg) @ W_prod + sc
        node_feats_list.append(node_feats)
    return jnp.concatenate(node_feats_list, axis=-1)


# speedup vs baseline: 2.3603x; 2.3603x over previous
import jax
import jax.numpy as jnp
import numpy as np
from jax import lax
from jax.experimental import pallas as pl
from jax.experimental.pallas import tpu as pltpu
from jax.experimental.pallas import tpu_sc as plsc

N = 10000
E = 320000
N_ELEM = 10
HID = 128
N_BESSEL = 8
N_SH = 9
R_MAX = 5.0
AVG_NEIGH = 32.0

C_NODES = 512
NBITS = 9
N_PAD = 10240
NBK = 10
NTILE = 16
EPT = E // NTILE
LCAP = 20480
SCH = 2048
GB = 16
FW = N_SH * HID
RPT = C_NODES // NTILE


def _geom_kernel(vec_ref, ea_ref, ef_ref):
    v = vec_ref[...]
    x, y, z = v[0:1, :], v[1:2, :], v[2:3, :]
    l2 = x * x + y * y + z * z
    lengths = jnp.sqrt(l2)
    inv = 1.0 / (lengths + 1e-9)
    ux, uy, uz = x * inv, y * inv, z * inv
    ea = jnp.concatenate([
        jnp.full_like(ux, 0.28209479177387814),
        0.4886025119029199 * uy,
        0.4886025119029199 * uz,
        0.4886025119029199 * ux,
        1.0925484305920792 * ux * uy,
        1.0925484305920792 * uy * uz,
        0.31539156525252005 * (3.0 * uz * uz - 1.0),
        1.0925484305920792 * ux * uz,
        0.5462742152960396 * (ux * ux - uy * uy),
        jnp.zeros_like(ux), jnp.zeros_like(ux), jnp.zeros_like(ux),
        jnp.zeros_like(ux), jnp.zeros_like(ux), jnp.zeros_like(ux),
        jnp.zeros_like(ux),
    ], axis=0)
    ea_ref[...] = ea
    pref = np.sqrt(2.0 / R_MAX)
    bess = jnp.concatenate(
        [pref * jnp.sin(float(k) * np.pi / R_MAX * lengths) * inv
         for k in range(1, N_BESSEL + 1)], axis=0)
    u_c = jnp.clip(lengths / R_MAX, 0.0, 1.0)
    p = 6.0
    env = (1.0 - ((p + 1.0) * (p + 2.0) / 2.0) * u_c ** 6
           + p * (p + 2.0) * u_c ** 7 - (p * (p + 1.0) / 2.0) * u_c ** 8)
    env = env * (u_c < 1.0).astype(jnp.float32)
    ef_ref[...] = bess * env


def _geometry(vectors_t):
    BE = 16000
    return pl.pallas_call(
        _geom_kernel,
        grid=(E // BE,),
        in_specs=[pl.BlockSpec((3, BE), lambda i: (0, i))],
        out_specs=[pl.BlockSpec((16, BE), lambda i: (0, i)),
                   pl.BlockSpec((N_BESSEL, BE), lambda i: (0, i))],
        out_shape=[jax.ShapeDtypeStruct((16, E), jnp.float32),
                   jax.ShapeDtypeStruct((N_BESSEL, E), jnp.float32)],
    )(vectors_t)


def _radial_kernel(eft_ref, w0_ref, w1_ref, rw0_ref, rw1_ref):
    ef = eft_ref[...]
    dn = (((0,), (0,)), ((), ()))
    rw0_ref[...] = jnp.tanh(lax.dot_general(ef, w0_ref[...], dn))
    rw1_ref[...] = jnp.tanh(lax.dot_general(ef, w1_ref[...], dn))


def _radial(eft, W_radial_0, W_radial_1):
    BE = 2560
    return pl.pallas_call(
        _radial_kernel,
        grid=(E // BE,),
        in_specs=[pl.BlockSpec((N_BESSEL, BE), lambda i: (0, i)),
                  pl.BlockSpec((N_BESSEL, HID), lambda i: (0, 0)),
                  pl.BlockSpec((N_BESSEL, HID), lambda i: (0, 0))],
        out_specs=[pl.BlockSpec((BE, HID), lambda i: (i, 0)),
                   pl.BlockSpec((BE, HID), lambda i: (i, 0))],
        out_shape=[jax.ShapeDtypeStruct((E, HID), jnp.float32),
                   jax.ShapeDtypeStruct((E, HID), jnp.float32)],
    )(eft, W_radial_0, W_radial_1)


def _embed_kernel(attrs_ref, w_ref, out_ref):
    out_ref[...] = attrs_ref[...] @ w_ref[...]


def _embed(node_attrs, W_embed):
    BN = 1000
    return pl.pallas_call(
        _embed_kernel,
        grid=(N // BN,),
        in_specs=[pl.BlockSpec((BN, N_ELEM), lambda i: (i, 0)),
                  pl.BlockSpec((N_ELEM, HID), lambda i: (0, 0))],
        out_specs=pl.BlockSpec((BN, HID), lambda i: (i, 0)),
        out_shape=jax.ShapeDtypeStruct((N, HID), jnp.float32),
    )(node_attrs, W_embed)


def _dense_kernel(neq_ref, attrs_ref, nf_ref, wmsg_ref, wprod_ref, wsc_ref,
                  out_ref):
    t = neq_ref[...] @ wmsg_ref[...]
    nf = nf_ref[...]
    attrs = attrs_ref[...]
    sc = attrs[:, 0:1] * (nf @ wsc_ref[0])
    for i in range(1, N_ELEM):
        sc = sc + attrs[:, i:i + 1] * (nf @ wsc_ref[i])
    out_ref[...] = t @ wprod_ref[...] + sc


def _dense(neq, node_attrs, nf, W_msg_scaled, W_prod, W_sc):
    BN = 1000
    return pl.pallas_call(
        _dense_kernel,
        grid=(N // BN,),
        in_specs=[pl.BlockSpec((BN, FW), lambda i: (i, 0)),
                  pl.BlockSpec((BN, N_ELEM), lambda i: (i, 0)),
                  pl.BlockSpec((BN, HID), lambda i: (i, 0)),
                  pl.BlockSpec((FW, HID), lambda i: (0, 0)),
                  pl.BlockSpec((HID, HID), lambda i: (0, 0)),
                  pl.BlockSpec((N_ELEM, HID, HID), lambda i: (0, 0, 0))],
        out_specs=pl.BlockSpec((BN, HID), lambda i: (i, 0)),
        out_shape=jax.ShapeDtypeStruct((N, HID), jnp.float32),
    )(neq, node_attrs, nf, W_msg_scaled, W_prod, W_sc)


def _bucket_body(src_hbm, dst_hbm, eidL, srcL, dstL, cnts,
                 srcb, dstb, keid, ksrc, kdst, cntb):
    cid = lax.axis_index("c")
    sid = lax.axis_index("s")
    t0 = sid * EPT
    pltpu.sync_copy(src_hbm.at[pl.ds(t0, EPT)], srcb)
    pltpu.sync_copy(dst_hbm.at[pl.ds(t0, EPT)], dstb)

    def bk_body(bi, cvec):
        b = 2 * bi + cid

        def comp_body(g, off):
            dstv = dstb[pl.ds(g * 16, 16)]
            srcv = srcb[pl.ds(g * 16, 16)]
            inb = lax.shift_right_logical(dstv, NBITS) == b
            eidv = t0 + g * 16 + lax.iota(jnp.int32, 16)
            pos = off + plsc.cumsum(inb.astype(jnp.int32)) - 1
            plsc.store_scatter(keid, [pos], eidv, mask=inb)
            plsc.store_scatter(ksrc, [pos], srcv, mask=inb)
            plsc.store_scatter(kdst, [pos], dstv, mask=inb)
            return off + jnp.sum(inb.astype(jnp.int32))
        n = lax.fori_loop(0, EPT // 16, comp_body, 0)

        seg = (bi * NTILE + sid) * LCAP
        nch = (n + SCH - 1) // SCH

        def cp_body(q, _):
            o = q * SCH
            pltpu.sync_copy(keid.at[pl.ds(o, SCH)],
                            eidL.at[cid].at[pl.ds(seg + o, SCH)])
            pltpu.sync_copy(ksrc.at[pl.ds(o, SCH)],
                            srcL.at[cid].at[pl.ds(seg + o, SCH)])
            pltpu.sync_copy(kdst.at[pl.ds(o, SCH)],
                            dstL.at[cid].at[pl.ds(seg + o, SCH)])
            return 0
        lax.fori_loop(0, nch, cp_body, 0)
        lane = lax.iota(jnp.int32, 16) == bi
        return jnp.where(lane, n, cvec)

    cvec = lax.fori_loop(0, NBK, bk_body, jnp.zeros((16,), jnp.int32))
    cntb[pl.ds(0, 16)] = cvec
    pltpu.sync_copy(cntb, cnts.at[cid].at[sid])


def _sc_bucket(src, dst):
    mesh = plsc.VectorSubcoreMesh(core_axis_name="c", subcore_axis_name="s")
    f = pl.kernel(
        _bucket_body,
        mesh=mesh,
        compiler_params=pltpu.CompilerParams(needs_layout_passes=False),
        out_type=[
            jax.ShapeDtypeStruct((2, NBK * NTILE * LCAP), jnp.int32),
            jax.ShapeDtypeStruct((2, NBK * NTILE * LCAP), jnp.int32),
            jax.ShapeDtypeStruct((2, NBK * NTILE * LCAP), jnp.int32),
            jax.ShapeDtypeStruct((2, NTILE, 16), jnp.int32),
        ],
        scratch_types=[
            pltpu.VMEM((EPT,), jnp.int32),
            pltpu.VMEM((EPT,), jnp.int32),
            pltpu.VMEM((EPT,), jnp.int32),
            pltpu.VMEM((EPT,), jnp.int32),
            pltpu.VMEM((EPT,), jnp.int32),
            pltpu.VMEM((16,), jnp.int32),
        ],
    )
    return f(src, dst)


def _accum_body(eidL, srcL, dstL, cnts, ea_hbm, rw_hbm, nf_hbm, neq_hbm,
                accT, cntv, eidc, srcc, dstc, keid, ksrc, kloc,
                gbuf, sidx, rwb, nfb, eab, semg):
    cid = lax.axis_index("c")
    sid = lax.axis_index("s")
    pltpu.sync_copy(cnts.at[cid], cntv)

    def z16(i, _):
        z = jnp.zeros((16,), jnp.int32)
        keid[pl.ds(i * 16, 16)] = z
        ksrc[pl.ds(i * 16, 16)] = z
        kloc[pl.ds(i * 16, 16)] = z
        return 0
    lax.fori_loop(0, (SCH + GB) // 16, z16, 0)

    def bk_body(bi, _c0):
        b = 2 * bi + cid
        base = b * C_NODES

        def zacc(i, _):
            r = i // (FW // 16)
            o = (i % (FW // 16)) * 16
            accT[r, pl.ds(o, 16)] = jnp.zeros((16,), jnp.float32)
            return 0
        lax.fori_loop(0, (RPT + 1) * FW // 16, zacc, 0)

        def seg_body(u, _c1):
            biv = jnp.full((16,), bi, jnp.int32)
            uv = jnp.full((16,), u, jnp.int32)
            nsegv = plsc.load_gather(cntv, [uv, biv])
            nseg = jnp.max(nsegv)
            seg = (bi * NTILE + u) * LCAP
            nch = (nseg + SCH - 1) // SCH

            def ch_body(q, _c2):
                o = q * SCH
                pltpu.sync_copy(eidL.at[cid].at[pl.ds(seg + o, SCH)], eidc)
                pltpu.sync_copy(srcL.at[cid].at[pl.ds(seg + o, SCH)], srcc)
                pltpu.sync_copy(dstL.at[cid].at[pl.ds(seg + o, SCH)], dstc)

                def comp_body(g, off):
                    dstv = dstc[pl.ds(g * 16, 16)]
                    loc = dstv - base
                    own = lax.shift_right_logical(loc, 5) == sid
                    pos16 = o + g * 16 + lax.iota(jnp.int32, 16)
                    keep = jnp.logical_and(own, pos16 < nseg)
                    pos = off + plsc.cumsum(keep.astype(jnp.int32)) - 1
                    plsc.store_scatter(keid, [pos], eidc[pl.ds(g * 16, 16)],
                                       mask=keep)
                    plsc.store_scatter(ksrc, [pos], srcc[pl.ds(g * 16, 16)],
                                       mask=keep)
                    plsc.store_scatter(kloc, [pos], loc & (RPT - 1), mask=keep)
                    return off + jnp.sum(keep.astype(jnp.int32))
                m = lax.fori_loop(0, SCH // 16, comp_body, 0)

                nbat = (m + GB - 1) // GB

                def bat_body(i, _c3):
                    k0 = i * GB
                    validv = k0 + lax.iota(jnp.int32, 16) < m
                    gbuf[pl.ds(0, 16)] = keid[pl.ds(k0, 16)]
                    sidx[pl.ds(0, 16)] = ksrc[pl.ds(k0, 16)]
                    c1 = pltpu.async_copy(rw_hbm.at[gbuf], rwb, semg)
                    c2 = pltpu.async_copy(ea_hbm.at[gbuf], eab, semg)
                    c3 = pltpu.async_copy(nf_hbm.at[sidx], nfb, semg)
                    c1.wait()
                    c2.wait()
                    c3.wait()
                    klocv = jnp.where(validv, kloc[pl.ds(k0, 16)], RPT)
                    for e in range(16):
                        row = klocv[e]
                        msg = []
                        for k in range(HID // 16):
                            msg.append(nfb[e, pl.ds(k * 16, 16)]
                                       * rwb[e, pl.ds(k * 16, 16)])
                        eav = eab[e, pl.ds(0, 16)]
                        for c in range(N_SH):
                            eac = eav[c]
                            for k in range(HID // 16):
                                plsc.addupdate(
                                    accT.at[row, pl.ds(c * HID + k * 16, 16)],
                                    msg[k] * eac)
                    return 0
                lax.fori_loop(0, nbat, bat_body, 0)
                return 0
            lax.fori_loop(0, nch, ch_body, 0)
            return 0
        lax.fori_loop(0, NTILE, seg_body, 0)

        r0 = base + sid * RPT

        def wb_body(q, _):
            pltpu.sync_copy(accT.at[pl.ds(q * 16, 16)],
                            neq_hbm.at[pl.ds(r0 + q * 16, 16)])
            return 0
        lax.fori_loop(0, RPT // 16, wb_body, 0)
        return 0
    lax.fori_loop(0, NBK, bk_body, 0)


def _sc_accum(eidL, srcL, dstL, cnts, eattr, rw, nf):
    mesh = plsc.VectorSubcoreMesh(core_axis_name="c", subcore_axis_name="s")
    f = pl.kernel(
        _accum_body,
        mesh=mesh,
        compiler_params=pltpu.CompilerParams(needs_layout_passes=False),
        out_type=jax.ShapeDtypeStruct((N_PAD, FW), jnp.float32),
        scratch_types=[
            pltpu.VMEM((RPT + 1, FW), jnp.float32),
            pltpu.VMEM((NTILE, 16), jnp.int32),
            pltpu.VMEM((SCH,), jnp.int32),
            pltpu.VMEM((SCH,), jnp.int32),
            pltpu.VMEM((SCH,), jnp.int32),
            pltpu.VMEM((SCH + GB,), jnp.int32),
            pltpu.VMEM((SCH + GB,), jnp.int32),
            pltpu.VMEM((SCH + GB,), jnp.int32),
            pltpu.VMEM((GB,), jnp.int32),
            pltpu.VMEM((GB,), jnp.int32),
            pltpu.VMEM((GB, HID), jnp.float32),
            pltpu.VMEM((GB, HID), jnp.float32),
            pltpu.VMEM((GB, HID), jnp.float32),
            pltpu.SemaphoreType.DMA,
        ],
    )
    return f(eidL, srcL, dstL, cnts, eattr, rw, nf)


def kernel(atom_pos, node_attrs, edge_index, shifts, W_embed, W_radial_0,
           W_radial_1, W_msg_0, W_msg_1, W_sc_0, W_sc_1, W_prod_0, W_prod_1):
    src = edge_index[0]
    dst = edge_index[1]
    vectors_t = (atom_pos[dst] - atom_pos[src] + shifts).T
    ea_t, ef_t = _geometry(vectors_t)
    eattr = jnp.pad(ea_t.T, ((0, 0), (0, HID - 16)))
    rw0, rw1 = _radial(ef_t, W_radial_0, W_radial_1)
    nf = _embed(node_attrs, W_embed)
    eidL, srcL, dstL, cnts = _sc_bucket(src, dst)
    outs = []
    for (rw, W_msg, W_sc, W_prod) in ((rw0, W_msg_0, W_sc_0, W_prod_0),
                                      (rw1, W_msg_1, W_sc_1, W_prod_1)):
        neq = _sc_accum(eidL, srcL, dstL, cnts, eattr, rw, nf)
        nf = _dense(neq[:N], node_attrs, nf, W_msg * (1.0 / AVG_NEIGH),
                    W_prod, W_sc)
        outs.append(nf)
    return jnp.concatenate(outs, axis=-1)
